# Initial kernel scaffold; baseline (speedup 1.0000x reference)
import jax
import jax.numpy as jnp
from jax.experimental import pallas as pl


def kernel(x):
    raise NotImplementedError("write your pallas kernel here")


# speedup vs baseline: 3.9638x; 3.9638x over previous
import jax
import jax.numpy as jnp
from jax.experimental import pallas as pl
from jax.experimental.pallas import tpu as pltpu

_SIZE = 20
_ROWS_BLOCK = 512


def _shl(a, k):
    return jnp.concatenate(
        [a[:, k:], jnp.zeros((a.shape[0], k), a.dtype)], axis=1
    )


def _ts_mean_kernel(x_ref, o_ref):
    x = x_ref[...]
    s2 = x + _shl(x, 1)
    s4 = s2 + _shl(s2, 2)
    s8 = s4 + _shl(s4, 4)
    s16 = s8 + _shl(s8, 8)
    s20 = s16 + _shl(s4, 16)
    t_out = o_ref.shape[1]
    o_ref[...] = s20[:, :t_out] * (1.0 / _SIZE)


def kernel(x):
    b, f, t = x.shape
    t_out = t - _SIZE + 1
    rows = b * f
    x2 = x.reshape(rows, t)
    grid = rows // _ROWS_BLOCK
    out = pl.pallas_call(
        _ts_mean_kernel,
        grid=(grid,),
        in_specs=[pl.BlockSpec((_ROWS_BLOCK, t), lambda i: (i, 0))],
        out_specs=pl.BlockSpec((_ROWS_BLOCK, t_out), lambda i: (i, 0)),
        out_shape=jax.ShapeDtypeStruct((rows, t_out), x.dtype),
        compiler_params=pltpu.CompilerParams(
            dimension_semantics=("parallel",),
        ),
    )(x2)
    return out.reshape(b, f, t_out)
